# baseline (speedup 1.0000x reference)
import functools

import jax
import jax.numpy as jnp
import numpy as np
from jax import lax
from jax.experimental import pallas as pl
from jax.experimental.pallas import tpu as pltpu
from jax.experimental.pallas import tpu_sc as plsc

F32 = jnp.float32
TW = 128


def _ln(x, g, b, eps=1e-5):
    m = jnp.mean(x, axis=-1, keepdims=True)
    v = jnp.mean((x - m) * (x - m), axis=-1, keepdims=True)
    return (x - m) * lax.rsqrt(v + eps) * g + b


def _proj_body(q_ref, k_ref, v_ref, xyz_ref, Wq_ref, bq_ref, gq_ref, btq_ref,
               Wk_ref, bk_ref, gk_ref, btk_ref, Wv_ref, bv_ref, Ww1_ref,
               tab_ref, q2_ref):
    query = jax.nn.relu(_ln(
        jnp.dot(q_ref[:], Wq_ref[:], preferred_element_type=F32) + bq_ref[:],
        gq_ref[:], btq_ref[:]))
    key = jax.nn.relu(_ln(
        jnp.dot(k_ref[:], Wk_ref[:], preferred_element_type=F32) + bk_ref[:],
        gk_ref[:], btk_ref[:]))
    value = jnp.dot(v_ref[:], Wv_ref[:], preferred_element_type=F32) + bv_ref[:]
    Ww1 = Ww1_ref[:]
    q2_ref[:] = jnp.dot(query, Ww1, preferred_element_type=F32)
    key2 = jnp.dot(key, Ww1, preferred_element_type=F32)
    B, C = value.shape
    G = key2.shape[1]
    pad = jnp.zeros((B, TW - C - G - 3), F32)
    tab_ref[:] = jnp.concatenate([value, key2, xyz_ref[:], pad], axis=-1)


def _run_proj(q, k, v, xyz, Wq, bq, gq, btq, Wk, bk, gk, btk, Wv, bv, Ww1):
    N, C = q.shape
    G = Ww1.shape[1]
    BA = 1000
    full2 = lambda a: pl.BlockSpec(a.shape, lambda i: (0, 0))
    return pl.pallas_call(
        _proj_body,
        grid=(N // BA,),
        in_specs=[
            pl.BlockSpec((BA, C), lambda i: (i, 0)),
            pl.BlockSpec((BA, C), lambda i: (i, 0)),
            pl.BlockSpec((BA, C), lambda i: (i, 0)),
            pl.BlockSpec((BA, 3), lambda i: (i, 0)),
            full2(Wq), full2(bq), full2(gq), full2(btq),
            full2(Wk), full2(bk), full2(gk), full2(btk),
            full2(Wv), full2(bv), full2(Ww1),
        ],
        out_specs=[
            pl.BlockSpec((BA, TW), lambda i: (i, 0)),
            pl.BlockSpec((BA, G), lambda i: (i, 0)),
        ],
        out_shape=[
            jax.ShapeDtypeStruct((N, TW), F32),
            jax.ShapeDtypeStruct((N, G), F32),
        ],
    )(q, k, v, xyz, Wq, bq, gq, btq, Wk, bk, gk, btk, Wv, bv, Ww1)


def _make_gather(NIDX):
    info = plsc.get_sparse_core_info()
    NC, NSUB = info.num_cores, info.num_subcores
    NW = NC * NSUB
    assert NIDX % NW == 0
    per_w = NIDX // NW
    CH = 200
    assert per_w % CH == 0 and CH % 8 == 0
    n_ch = per_w // CH
    mesh = plsc.VectorSubcoreMesh(core_axis_name="c", subcore_axis_name="s")

    @functools.partial(
        pl.kernel, mesh=mesh,
        out_type=jax.ShapeDtypeStruct((NIDX, TW), F32),
        scratch_types=[
            pltpu.VMEM((per_w,), jnp.int32),
            pltpu.VMEM((CH, TW), F32),
            pltpu.VMEM((CH, TW), F32),
            pltpu.SemaphoreType.DMA,
            pltpu.SemaphoreType.DMA,
        ],
    )
    def gather_k(tab_hbm, idx_hbm, otab, idx_v, rows0, rows1, sem0, sem1):
        wid = lax.axis_index("s") * NC + lax.axis_index("c")
        base0 = wid * per_w
        pltpu.sync_copy(idx_hbm.at[pl.ds(base0, per_w)], idx_v)

        def start_g(j, rv, sem):
            pltpu.async_copy(tab_hbm.at[idx_v.at[pl.ds(j * CH, CH)]], rv, sem)

        def wait_g(j, rv, sem):
            pltpu.make_async_copy(
                tab_hbm.at[idx_v.at[pl.ds(j * CH, CH)]], rv, sem).wait()

        def write(j, rv):
            pltpu.sync_copy(rv, otab.at[pl.ds(base0 + j * CH, CH)])

        start_g(0, rows0, sem0)

        def body(jj, carry):
            j0 = 2 * jj
            start_g(j0 + 1, rows1, sem1)
            wait_g(j0, rows0, sem0)
            write(j0, rows0)

            @pl.when(j0 + 2 < n_ch)
            def _():
                start_g(j0 + 2, rows0, sem0)

            wait_g(j0 + 1, rows1, sem1)
            write(j0 + 1, rows1)
            return carry

        lax.fori_loop(0, n_ch // 2, body, 0)
        if n_ch % 2 == 1:
            wait_g(n_ch - 1, rows0, sem0)
            write(n_ch - 1, rows0)

    return gather_k


def _dot(a, b):
    return jnp.dot(a, b, preferred_element_type=F32)


def _attn_body(gtab_ref, q2_ref, xyz_ref, idx_ref,
               Xsel_ref, P3_ref, W1mid_ref, bp1t_ref, M64p_ref, gpt_ref,
               btpt_ref, W2f_ref, ct_ref, K2sel_ref, P8_ref, Mseg_ref,
               gwt_ref, btwt_ref, Ww2b_ref, bw2t_ref, Sg_ref, E2_ref,
               EB2_ref, SB_ref, out_ref, *, B, NS, C, G):
    eps = 1e-5
    bf = jnp.bfloat16
    gtabB = gtab_ref[:].reshape(B, NS * TW)
    pos = _dot(gtabB, Xsel_ref[:]) - _dot(xyz_ref[:], P3_ref[:])
    hpre = _dot(pos, W1mid_ref[:]) + bp1t_ref[:]
    h8p = hpre.reshape(B * G, TW)
    mb = _dot(h8p.astype(bf), M64p_ref[:])
    m2b = _dot((h8p * h8p).astype(bf), M64p_ref[:])
    xn = (h8p - mb) * lax.rsqrt(m2b - mb * mb + eps)
    h8 = jax.nn.relu(xn * gpt_ref[:] + btpt_ref[:])
    pp = _dot(h8.astype(bf), W2f_ref[:])
    A = gtabB + pp.reshape(B, NS * TW) + ct_ref[:]
    wpre = _dot(A.astype(bf), K2sel_ref[:]) - _dot(q2_ref[:], P8_ref[:])
    mbw = _dot(wpre, Mseg_ref[:])
    m2bw = _dot(wpre * wpre, Mseg_ref[:])
    tn = (wpre - mbw) * lax.rsqrt(m2bw - mbw * mbw + eps)
    t = jax.nn.relu(tn * gwt_ref[:] + btwt_ref[:])
    w = _dot(t, Ww2b_ref[:]) + bw2t_ref[:]
    m = jnp.max(w, axis=-1, keepdims=True)
    e = jnp.exp(w - m)
    w3 = e / _dot(e, Sg_ref[:])
    maskf = jnp.sign(idx_ref[:] + 1).astype(F32)
    w3 = w3 * _dot(maskf, E2_ref[:])
    wfull = _dot(w3.astype(bf), EB2_ref[:])
    out_ref[:] = _dot((wfull * A).astype(bf), SB_ref[:])


def _run_attn(gtab, q2, xyz, idx, consts, N, NS, C, G):
    BC = 400
    R = BC * NS
    fullb = lambda a: pl.BlockSpec(a.shape, lambda i: tuple(0 for _ in a.shape))
    body = functools.partial(_attn_body, B=BC, NS=NS, C=C, G=G)
    return pl.pallas_call(
        body,
        grid=(N // BC,),
        in_specs=[
            pl.BlockSpec((R, TW), lambda i: (i, 0)),
            pl.BlockSpec((BC, G), lambda i: (i, 0)),
            pl.BlockSpec((BC, 3), lambda i: (i, 0)),
            pl.BlockSpec((BC, NS), lambda i: (i, 0)),
        ] + [fullb(c) for c in consts],
        out_specs=pl.BlockSpec((BC, C), lambda i: (i, 0)),
        out_shape=jax.ShapeDtypeStruct((N, C), F32),
    )(gtab, q2, xyz, idx, *consts)


def kernel(q, k, v, xyz, reference_index, Wq, bq, gq, btq, Wk, bk, gk, btk,
           Wv, bv, Wp1, bp1, gp, btp, Wp2, bp2, Ww1, bw1, gw, btw, Ww2, bw2):
    N, C = q.shape
    NS = reference_index.shape[1]
    G = Ww1.shape[1]

    r2 = lambda a: a.reshape(1, -1)
    tile16 = lambda a: jnp.tile(a.reshape(1, -1), (1, NS))
    eyeNS = np.eye(NS, dtype=np.float32)
    cw = bp2 @ Ww1 + bw1
    Wp2c = jnp.concatenate(
        [Wp2, Wp2 @ Ww1, jnp.zeros((C, TW - C - G), F32)], axis=1)
    c128 = jnp.concatenate([bp2, cw, jnp.zeros((TW - C - G,), F32)])
    bf = jnp.bfloat16
    Xblk = np.zeros((TW, G), np.float32)
    Xblk[C + G:C + G + 3, 0:3] = np.eye(3)
    Xsel = np.kron(eyeNS, Xblk)
    P3 = np.tile(np.hstack([np.eye(3, dtype=np.float32),
                            np.zeros((3, G - 3), np.float32)]), (1, NS))
    Kblk = np.zeros((TW, G), np.float32)
    Kblk[C:C + G] = np.eye(G)
    K2sel = np.kron(eyeNS, Kblk).astype(bf)
    M64p = np.kron(np.eye(2, dtype=np.float32),
                   np.full((C, C), 1.0 / C, np.float32)).astype(bf)
    Mseg = np.kron(eyeNS, np.full((G, G), 1.0 / G, np.float32))
    P8 = np.tile(np.eye(G, dtype=np.float32), (1, NS))
    Sg = np.kron(np.ones((NS, NS), np.float32), np.eye(G, dtype=np.float32))
    E2 = np.kron(eyeNS, np.ones((1, G), np.float32))
    EBblk = np.zeros((G, TW), np.float32)
    for g in range(G):
        EBblk[g, g * (C // G):(g + 1) * (C // G)] = 1.0
    EB2 = np.kron(eyeNS, EBblk).astype(bf)
    SB = np.tile(np.vstack([np.eye(C, dtype=np.float32),
                            np.zeros((TW - C, C), np.float32)]),
                 (NS, 1)).astype(bf)
    W1j = jnp.zeros((G, C), F32).at[0:3].set(Wp1)
    W1mid = jnp.kron(jnp.eye(NS, dtype=F32), W1j)
    W2f = jnp.kron(jnp.eye(2, dtype=F32), Wp2c).astype(bf)
    Ww2b = jnp.kron(jnp.eye(NS, dtype=F32), Ww2)
    tile2 = lambda a: jnp.tile(a.reshape(1, -1), (1, 2))
    consts = [Xsel, P3, W1mid, tile16(bp1), M64p, tile2(gp), tile2(btp),
              W2f, tile16(c128), K2sel, P8, Mseg, tile16(gw), tile16(btw),
              Ww2b, tile16(bw2), Sg, E2, EB2, SB]

    table, q2 = _run_proj(
        q, k, v, xyz, Wq, r2(bq), r2(gq), r2(btq),
        Wk, r2(bk), r2(gk), r2(btk), Wv, r2(bv), Ww1)

    idx32 = reference_index.astype(jnp.int32)
    P = 5
    NP = N // P
    gather = _make_gather(NP * NS)
    outs = []
    for p in range(P):
        sl = slice(p * NP, (p + 1) * NP)
        gtab_p = gather(table, idx32[sl].reshape(-1))
        outs.append(_run_attn(gtab_p, q2[sl], xyz[sl], idx32[sl],
                              consts, NP, NS, C, G))
    return jnp.concatenate(outs, axis=0)


# speedup vs baseline: 1.0862x; 1.0862x over previous
import functools

import jax
import jax.numpy as jnp
import numpy as np
from jax import lax
from jax.experimental import pallas as pl
from jax.experimental.pallas import tpu as pltpu
from jax.experimental.pallas import tpu_sc as plsc

F32 = jnp.float32
TW = 128


def _ln(x, g, b, eps=1e-5):
    m = jnp.mean(x, axis=-1, keepdims=True)
    v = jnp.mean((x - m) * (x - m), axis=-1, keepdims=True)
    return (x - m) * lax.rsqrt(v + eps) * g + b


def _proj_body(q_ref, k_ref, v_ref, xyz_ref, Wqk_ref, bqk_ref, M2_ref,
               g2_ref, bt2_ref, Wv128_ref, WW1x_ref, X3_ref, WW1q_ref,
               bias128_ref, tab_ref, q2_ref):
    eps = 1e-5
    bf = jnp.bfloat16
    qk = jnp.concatenate([q_ref[:], k_ref[:]], axis=1)
    qkp = _dot(qk.astype(bf), Wqk_ref[:]) + bqk_ref[:]
    mb = _dot(qkp.astype(bf), M2_ref[:])
    m2b = _dot((qkp * qkp).astype(bf), M2_ref[:])
    xn = (qkp - mb) * lax.rsqrt(m2b - mb * mb + eps)
    qkn = jax.nn.relu(xn * g2_ref[:] + bt2_ref[:])
    tab_ref[:] = (_dot(v_ref[:].astype(bf), Wv128_ref[:])
                  + _dot(qkn.astype(bf), WW1x_ref[:])
                  + _dot(xyz_ref[:], X3_ref[:]) + bias128_ref[:])
    q2_ref[:] = _dot(qkn.astype(bf), WW1q_ref[:])


def _run_proj(q, k, v, xyz, Wq, bq, gq, btq, Wk, bk, gk, btk, Wv, bv, Ww1):
    N, C = q.shape
    G = Ww1.shape[1]
    BA = 1000
    bf = jnp.bfloat16
    cc = lambda *xs: jnp.concatenate([x.reshape(1, -1) for x in xs], axis=1)
    Wqk = jax.scipy.linalg.block_diag(Wq, Wk).astype(bf)
    bqk = cc(bq, bk)
    M2 = np.kron(np.eye(2, dtype=np.float32),
                 np.full((C, C), 1.0 / C, np.float32)).astype(bf)
    g2, bt2 = cc(gq, gk), cc(btq, btk)
    Wv128 = jnp.concatenate(
        [Wv, jnp.zeros((C, TW - C), F32)], axis=1).astype(bf)
    WW1x = jnp.zeros((TW, TW), F32).at[C:2 * C, C:C + G].set(Ww1).astype(bf)
    X3 = np.zeros((3, TW), np.float32)
    X3[:, C + G:C + G + 3] = np.eye(3)
    WW1q = jnp.zeros((TW, G), F32).at[0:C].set(Ww1).astype(bf)
    bias128 = jnp.concatenate([bv, jnp.zeros((TW - C,), F32)]).reshape(1, -1)
    full2 = lambda a: pl.BlockSpec(a.shape, lambda i: (0, 0))
    consts = [Wqk, bqk, M2, g2, bt2, Wv128, WW1x, X3, WW1q, bias128]
    return pl.pallas_call(
        _proj_body,
        grid=(N // BA,),
        in_specs=[
            pl.BlockSpec((BA, C), lambda i: (i, 0)),
            pl.BlockSpec((BA, C), lambda i: (i, 0)),
            pl.BlockSpec((BA, C), lambda i: (i, 0)),
            pl.BlockSpec((BA, 3), lambda i: (i, 0)),
        ] + [full2(c) for c in consts],
        out_specs=[
            pl.BlockSpec((BA, TW), lambda i: (i, 0)),
            pl.BlockSpec((BA, G), lambda i: (i, 0)),
        ],
        out_shape=[
            jax.ShapeDtypeStruct((N, TW), F32),
            jax.ShapeDtypeStruct((N, G), F32),
        ],
    )(q, k, v, xyz, *consts)


def _make_gather(NIDX):
    info = plsc.get_sparse_core_info()
    NC, NSUB = info.num_cores, info.num_subcores
    NW = NC * NSUB
    assert NIDX % NW == 0
    per_w = NIDX // NW
    CH = 1000
    assert per_w % CH == 0 and CH % 8 == 0
    n_ch = per_w // CH
    mesh = plsc.VectorSubcoreMesh(core_axis_name="c", subcore_axis_name="s")

    @functools.partial(
        pl.kernel, mesh=mesh,
        out_type=jax.ShapeDtypeStruct((NIDX, TW), F32),
        scratch_types=[
            pltpu.VMEM((CH,), jnp.int32),
            pltpu.VMEM((CH, TW), F32),
            pltpu.SemaphoreType.DMA,
        ],
    )
    def gather_k(tab_hbm, idx_hbm, otab, idx_v, rows_v, sem):
        wid = lax.axis_index("s") * NC + lax.axis_index("c")
        base0 = wid * per_w

        def body(j, carry):
            base = base0 + j * CH
            pltpu.sync_copy(idx_hbm.at[pl.ds(base, CH)], idx_v)
            pltpu.async_copy(tab_hbm.at[idx_v], rows_v, sem).wait()
            pltpu.sync_copy(rows_v, otab.at[pl.ds(base, CH)])
            return carry

        lax.fori_loop(0, n_ch, body, 0)

    return gather_k


def _dot(a, b):
    return jnp.dot(a, b, preferred_element_type=F32)


def _attn_body(gtab_ref, q2_ref, xyz_ref, idx_ref,
               Xsel_ref, P3_ref, W1mid_ref, bp1t_ref, M64p_ref, gpt_ref,
               btpt_ref, W2f_ref, ct_ref, K2sel_ref, P8_ref, Mseg_ref,
               gwt_ref, btwt_ref, Ww2b_ref, bw2t_ref, Sg_ref, E2_ref,
               EB2_ref, SB_ref, out_ref, *, B, NS, C, G):
    eps = 1e-5
    bf = jnp.bfloat16
    gtabB = gtab_ref[:].reshape(B, NS * TW)
    pos = _dot(gtabB, Xsel_ref[:]) - _dot(xyz_ref[:], P3_ref[:])
    hpre = _dot(pos, W1mid_ref[:]) + bp1t_ref[:]
    h8p = hpre.reshape(B * G, TW)
    mb = _dot(h8p.astype(bf), M64p_ref[:])
    m2b = _dot((h8p * h8p).astype(bf), M64p_ref[:])
    xn = (h8p - mb) * lax.rsqrt(m2b - mb * mb + eps)
    h8 = jax.nn.relu(xn * gpt_ref[:] + btpt_ref[:])
    pp = _dot(h8.astype(bf), W2f_ref[:])
    A = gtabB + pp.reshape(B, NS * TW) + ct_ref[:]
    wpre = _dot(A.astype(bf), K2sel_ref[:]) - _dot(q2_ref[:], P8_ref[:])
    mbw = _dot(wpre, Mseg_ref[:])
    m2bw = _dot(wpre * wpre, Mseg_ref[:])
    tn = (wpre - mbw) * lax.rsqrt(m2bw - mbw * mbw + eps)
    t = jax.nn.relu(tn * gwt_ref[:] + btwt_ref[:])
    w = _dot(t, Ww2b_ref[:]) + bw2t_ref[:]
    m = jnp.max(w, axis=-1, keepdims=True)
    e = jnp.exp(w - m)
    w3 = e / _dot(e, Sg_ref[:])
    maskf = jnp.sign(idx_ref[:] + 1).astype(F32)
    w3 = w3 * _dot(maskf, E2_ref[:])
    wfull = _dot(w3.astype(bf), EB2_ref[:])
    out_ref[:] = _dot((wfull * A).astype(bf), SB_ref[:])


def _run_attn(gtab, q2, xyz, idx, consts, N, NS, C, G):
    BC = 1000
    R = BC * NS
    fullb = lambda a: pl.BlockSpec(a.shape, lambda i: tuple(0 for _ in a.shape))
    body = functools.partial(_attn_body, B=BC, NS=NS, C=C, G=G)
    return pl.pallas_call(
        body,
        grid=(N // BC,),
        in_specs=[
            pl.BlockSpec((R, TW), lambda i: (i, 0)),
            pl.BlockSpec((BC, G), lambda i: (i, 0)),
            pl.BlockSpec((BC, 3), lambda i: (i, 0)),
            pl.BlockSpec((BC, NS), lambda i: (i, 0)),
        ] + [fullb(c) for c in consts],
        out_specs=pl.BlockSpec((BC, C), lambda i: (i, 0)),
        out_shape=jax.ShapeDtypeStruct((N, C), F32),
    )(gtab, q2, xyz, idx, *consts)


def kernel(q, k, v, xyz, reference_index, Wq, bq, gq, btq, Wk, bk, gk, btk,
           Wv, bv, Wp1, bp1, gp, btp, Wp2, bp2, Ww1, bw1, gw, btw, Ww2, bw2):
    N, C = q.shape
    NS = reference_index.shape[1]
    G = Ww1.shape[1]

    r2 = lambda a: a.reshape(1, -1)
    tile16 = lambda a: jnp.tile(a.reshape(1, -1), (1, NS))
    eyeNS = np.eye(NS, dtype=np.float32)
    cw = bp2 @ Ww1 + bw1
    Wp2c = jnp.concatenate(
        [Wp2, Wp2 @ Ww1, jnp.zeros((C, TW - C - G), F32)], axis=1)
    c128 = jnp.concatenate([bp2, cw, jnp.zeros((TW - C - G,), F32)])
    bf = jnp.bfloat16
    Xblk = np.zeros((TW, G), np.float32)
    Xblk[C + G:C + G + 3, 0:3] = np.eye(3)
    Xsel = np.kron(eyeNS, Xblk)
    P3 = np.tile(np.hstack([np.eye(3, dtype=np.float32),
                            np.zeros((3, G - 3), np.float32)]), (1, NS))
    Kblk = np.zeros((TW, G), np.float32)
    Kblk[C:C + G] = np.eye(G)
    K2sel = np.kron(eyeNS, Kblk).astype(bf)
    M64p = np.kron(np.eye(2, dtype=np.float32),
                   np.full((C, C), 1.0 / C, np.float32)).astype(bf)
    Mseg = np.kron(eyeNS, np.full((G, G), 1.0 / G, np.float32))
    P8 = np.tile(np.eye(G, dtype=np.float32), (1, NS))
    Sg = np.kron(np.ones((NS, NS), np.float32), np.eye(G, dtype=np.float32))
    E2 = np.kron(eyeNS, np.ones((1, G), np.float32))
    EBblk = np.zeros((G, TW), np.float32)
    for g in range(G):
        EBblk[g, g * (C // G):(g + 1) * (C // G)] = 1.0
    EB2 = np.kron(eyeNS, EBblk).astype(bf)
    SB = np.tile(np.vstack([np.eye(C, dtype=np.float32),
                            np.zeros((TW - C, C), np.float32)]),
                 (NS, 1)).astype(bf)
    W1j = jnp.zeros((G, C), F32).at[0:3].set(Wp1)
    W1mid = jnp.kron(jnp.eye(NS, dtype=F32), W1j)
    W2f = jnp.kron(jnp.eye(2, dtype=F32), Wp2c).astype(bf)
    Ww2b = jnp.kron(jnp.eye(NS, dtype=F32), Ww2)
    tile2 = lambda a: jnp.tile(a.reshape(1, -1), (1, 2))
    consts = [Xsel, P3, W1mid, tile16(bp1), M64p, tile2(gp), tile2(btp),
              W2f, tile16(c128), K2sel, P8, Mseg, tile16(gw), tile16(btw),
              Ww2b, tile16(bw2), Sg, E2, EB2, SB]

    table, q2 = _run_proj(
        q, k, v, xyz, Wq, bq, gq, btq, Wk, bk, gk, btk, Wv, bv, Ww1)

    idx32 = reference_index.astype(jnp.int32)
    P = 5
    NP = N // P
    gather = _make_gather(NP * NS)
    outs = []
    for p in range(P):
        sl = slice(p * NP, (p + 1) * NP)
        gtab_p = gather(table, idx32[sl].reshape(-1))
        outs.append(_run_attn(gtab_p, q2[sl], xyz[sl], idx32[sl],
                              consts, NP, NS, C, G))
    return jnp.concatenate(outs, axis=0)
